# baseline (speedup 1.0000x reference)
import functools

import jax
import jax.numpy as jnp
from jax import lax
from jax.experimental import pallas as pl
from jax.experimental.pallas import tpu as pltpu
from jax.experimental.pallas import tpu_sc as plsc

BATCH = 4096
SEQ_LEN = 200
DIM = 64
PDIM = 128

NUM_CORES = 2
NUM_SUBCORES = 16
NUM_WORKERS = NUM_CORES * NUM_SUBCORES

TOTAL = BATCH * SEQ_LEN
PER_WORKER = TOTAL // NUM_WORKERS
GWIDTH = 128
CHUNK = 256
NCHUNK = PER_WORKER // CHUNK
NPAIR = NCHUNK // 2

NUM_EMB = 1000000


@functools.partial(
    pl.kernel,
    mesh=plsc.VectorSubcoreMesh(core_axis_name="c", subcore_axis_name="s"),
    out_type=jax.ShapeDtypeStruct((TOTAL, PDIM), jnp.float32),
    scratch_types=[
        pltpu.VMEM((2 * NCHUNK, GWIDTH), jnp.int32),
        pltpu.VMEM((CHUNK, PDIM), jnp.float32),
        pltpu.VMEM((CHUNK, PDIM), jnp.float32),
        pltpu.SemaphoreType.DMA,
        pltpu.SemaphoreType.DMA,
        pltpu.SemaphoreType.DMA,
        pltpu.SemaphoreType.DMA,
    ],
)
def _gather_kernel(idx_hbm, table_hbm, out_hbm, idx_v, buf_a, buf_b,
                   gs_a, gs_b, ws_a, ws_b):
    wid = lax.axis_index("s") * NUM_CORES + lax.axis_index("c")
    pltpu.sync_copy(idx_hbm.at[wid], idx_v)
    base = wid * PER_WORKER

    def gather(c, buf, sem):
        for h in range(2):
            pltpu.async_copy(
                table_hbm.at[idx_v.at[2 * c + h]],
                buf.at[pl.ds(h * GWIDTH, GWIDTH)],
                sem,
            )

    def write(c, buf, sem):
        return pltpu.async_copy(
            buf, out_hbm.at[pl.ds(base + c * CHUNK, CHUNK)], sem
        )

    def wait_gather(buf, sem):
        for h in range(2):
            pltpu.make_async_copy(
                table_hbm.at[idx_v.at[0]],
                buf.at[pl.ds(h * GWIDTH, GWIDTH)],
                sem,
            ).wait()

    def wait_write(buf, sem):
        pltpu.make_async_copy(buf, out_hbm.at[pl.ds(0, CHUNK)], sem).wait()

    gather(0, buf_a, gs_a)

    def pair(k, carry):
        c0 = 2 * k
        wait_gather(buf_a, gs_a)

        @pl.when(k > 0)
        def _():
            wait_write(buf_b, ws_b)

        write(c0, buf_a, ws_a)
        gather(c0 + 1, buf_b, gs_b)
        wait_gather(buf_b, gs_b)
        wait_write(buf_a, ws_a)
        write(c0 + 1, buf_b, ws_b)

        @pl.when(k < NPAIR - 1)
        def _():
            gather(c0 + 2, buf_a, gs_a)

        return carry

    lax.fori_loop(0, NPAIR, pair, 0)
    wait_write(buf_b, ws_b)


def kernel(token_ids, weight):
    wpad = jnp.pad(weight, ((0, 0), (0, PDIM - DIM)))
    flat_idx = token_ids.reshape(NUM_WORKERS, 2 * NCHUNK, GWIDTH)
    out = _gather_kernel(flat_idx, wpad)
    return out[:, :DIM].reshape(BATCH, SEQ_LEN, DIM)


# speedup vs baseline: 1.0007x; 1.0007x over previous
import functools

import jax
import jax.numpy as jnp
from jax import lax
from jax.experimental import pallas as pl
from jax.experimental.pallas import tpu as pltpu
from jax.experimental.pallas import tpu_sc as plsc

BATCH = 4096
SEQ_LEN = 200
DIM = 64
PDIM = 128

NUM_CORES = 2
NUM_SUBCORES = 16
NUM_WORKERS = NUM_CORES * NUM_SUBCORES

TOTAL = BATCH * SEQ_LEN
PER_WORKER = TOTAL // NUM_WORKERS
GWIDTH = 128
CHUNK = 128
NSUB = CHUNK // GWIDTH
NCHUNK = PER_WORKER // CHUNK
NBUF = 4
NROT = NCHUNK // NBUF

NUM_EMB = 1000000


@functools.partial(
    pl.kernel,
    mesh=plsc.VectorSubcoreMesh(core_axis_name="c", subcore_axis_name="s"),
    out_type=jax.ShapeDtypeStruct((TOTAL, PDIM), jnp.float32),
    scratch_types=[
        pltpu.VMEM((NSUB * NCHUNK, GWIDTH), jnp.int32),
        [pltpu.VMEM((CHUNK, PDIM), jnp.float32) for _ in range(NBUF)],
        [pltpu.SemaphoreType.DMA for _ in range(NBUF)],
        [pltpu.SemaphoreType.DMA for _ in range(NBUF)],
    ],
)
def _gather_kernel(idx_hbm, table_hbm, out_hbm, idx_v, bufs, gsems, wsems):
    wid = lax.axis_index("s") * NUM_CORES + lax.axis_index("c")
    pltpu.sync_copy(idx_hbm.at[wid], idx_v)
    base = wid * PER_WORKER

    def gather(c, buf, sem):
        for h in range(NSUB):
            pltpu.async_copy(
                table_hbm.at[idx_v.at[NSUB * c + h]],
                buf.at[pl.ds(h * GWIDTH, GWIDTH)],
                sem,
            )

    def write(c, buf, sem):
        return pltpu.async_copy(
            buf, out_hbm.at[pl.ds(base + c * CHUNK, CHUNK)], sem
        )

    def wait_gather(buf, sem):
        for h in range(NSUB):
            pltpu.make_async_copy(
                table_hbm.at[idx_v.at[0]],
                buf.at[pl.ds(h * GWIDTH, GWIDTH)],
                sem,
            ).wait()

    def wait_write(buf, sem):
        pltpu.make_async_copy(buf, out_hbm.at[pl.ds(0, CHUNK)], sem).wait()

    for j in range(NBUF):
        gather(j, bufs[j], gsems[j])

    def rot(k, carry):
        c0 = NBUF * k
        for j in range(NBUF):
            wait_gather(bufs[j], gsems[j])
            write(c0 + j, bufs[j], wsems[j])
        for j in range(NBUF):
            wait_write(bufs[j], wsems[j])

            @pl.when(k < NROT - 1)
            def _(j=j):
                gather(c0 + NBUF + j, bufs[j], gsems[j])

        return carry

    lax.fori_loop(0, NROT, rot, 0)


def kernel(token_ids, weight):
    wpad = jnp.pad(weight, ((0, 0), (0, PDIM - DIM)))
    flat_idx = token_ids.reshape(NUM_WORKERS, NSUB * NCHUNK, GWIDTH)
    out = _gather_kernel(flat_idx, wpad)
    return out[:, :DIM].reshape(BATCH, SEQ_LEN, DIM)
